# baseline (speedup 1.0000x reference)
import functools

import jax
import jax.numpy as jnp
from jax import lax
from jax.experimental import pallas as pl
from jax.experimental.pallas import tpu as pltpu
from jax.experimental.pallas import tpu_sc as plsc

EMB = 64
HID = 256
TAGS = 20
B = 4096
L = 200
HALF = L // 2
LANES = 16
NC = 2
NS = 16
NW = NC * NS
SPW = B // NW


def _embedding_bag_sc(x2, emb):
    mesh = plsc.VectorSubcoreMesh(core_axis_name="c", subcore_axis_name="s")

    @functools.partial(
        pl.kernel,
        out_type=jax.ShapeDtypeStruct((B, EMB), jnp.float32),
        mesh=mesh,
        compiler_params=pltpu.CompilerParams(use_tc_tiling_on_sc=False),
        scratch_types=[
            pltpu.VMEM((2 * SPW, HALF), jnp.int32),
            pltpu.VMEM((L, EMB), jnp.float32),
            pltpu.VMEM((SPW, EMB), jnp.float32),
            pltpu.SemaphoreType.DMA,
        ],
    )
    def bag(x_hbm, emb_hbm, out_hbm, idx_v, rows_v, out_v, sem):
        wid = lax.axis_index("s") * NC + lax.axis_index("c")
        base = wid * SPW
        pltpu.sync_copy(x_hbm.at[pl.ds(2 * base, 2 * SPW)], idx_v)

        def body(s, carry):
            h0 = pltpu.async_copy(
                emb_hbm.at[idx_v.at[2 * s]], rows_v.at[pl.ds(0, HALF)], sem)
            h1 = pltpu.async_copy(
                emb_hbm.at[idx_v.at[2 * s + 1]], rows_v.at[pl.ds(HALF, HALF)], sem)
            h0.wait()
            h1.wait()

            def rbody(r, accs):
                return tuple(
                    accs[k] + rows_v[r, pl.ds(LANES * k, LANES)] for k in range(4))

            zero = jnp.zeros((LANES,), jnp.float32)
            accs = lax.fori_loop(0, L, rbody, (zero,) * 4)
            for k in range(4):
                out_v[s, pl.ds(LANES * k, LANES)] = accs[k]
            return carry

        lax.fori_loop(0, SPW, body, 0)
        pltpu.sync_copy(out_v, out_hbm.at[pl.ds(base, SPW)])

    return bag(x2, emb)


def _mlp_tc(sums, W1, b1, W2, b2):
    blk = 512

    def mlp_body(s_ref, w1_ref, b1_ref, w2_ref, b2_ref, o_ref):
        xa = s_ref[...] * (1.0 / L)
        h = jnp.dot(xa, w1_ref[...], preferred_element_type=jnp.float32)
        h = jnp.maximum(h + b1_ref[...], 0.0)
        o_ref[...] = (
            jnp.dot(h, w2_ref[...], preferred_element_type=jnp.float32)
            + b2_ref[...])

    return pl.pallas_call(
        mlp_body,
        grid=(B // blk,),
        in_specs=[
            pl.BlockSpec((blk, EMB), lambda i: (i, 0)),
            pl.BlockSpec((EMB, HID), lambda i: (0, 0)),
            pl.BlockSpec((1, HID), lambda i: (0, 0)),
            pl.BlockSpec((HID, TAGS), lambda i: (0, 0)),
            pl.BlockSpec((1, TAGS), lambda i: (0, 0)),
        ],
        out_specs=pl.BlockSpec((blk, TAGS), lambda i: (i, 0)),
        out_shape=jax.ShapeDtypeStruct((B, TAGS), jnp.float32),
    )(sums, W1, b1, W2, b2)


def kernel(x, emb, W1, b1, W2, b2):
    x2 = x.reshape(2 * B, HALF)
    sums = _embedding_bag_sc(x2, emb)
    return _mlp_tc(sums, W1, b1.reshape(1, HID), W2, b2.reshape(1, TAGS))


# speedup vs baseline: 1.1705x; 1.1705x over previous
import functools

import jax
import jax.numpy as jnp
from jax import lax
from jax.experimental import pallas as pl
from jax.experimental.pallas import tpu as pltpu
from jax.experimental.pallas import tpu_sc as plsc

EMB = 64
HID = 256
TAGS = 20
B = 4096
L = 200
HALF = L // 2
LANES = 16
NC = 2
NS = 16
NW = NC * NS
SPW = B // NW


def _embedding_bag_sc(x2, emb):
    mesh = plsc.VectorSubcoreMesh(core_axis_name="c", subcore_axis_name="s")

    @functools.partial(
        pl.kernel,
        out_type=jax.ShapeDtypeStruct((B, EMB), jnp.float32),
        mesh=mesh,
        compiler_params=pltpu.CompilerParams(use_tc_tiling_on_sc=False),
        scratch_types=[
            pltpu.VMEM((2 * SPW, HALF), jnp.int32),
            pltpu.VMEM((2, L, EMB), jnp.float32),
            pltpu.VMEM((SPW, EMB), jnp.float32),
            pltpu.SemaphoreType.DMA,
            pltpu.SemaphoreType.DMA,
        ],
    )
    def bag(x_hbm, emb_hbm, out_hbm, idx_v, rows_v, out_v, sem0, sem1):
        wid = lax.axis_index("s") * NC + lax.axis_index("c")
        base = wid * SPW
        pltpu.sync_copy(x_hbm.at[pl.ds(2 * base, 2 * SPW)], idx_v)
        sems = (sem0, sem1)

        def issue(s, b):
            pltpu.async_copy(
                emb_hbm.at[idx_v.at[2 * s]], rows_v.at[b, pl.ds(0, HALF)], sems[b])
            pltpu.async_copy(
                emb_hbm.at[idx_v.at[2 * s + 1]],
                rows_v.at[b, pl.ds(HALF, HALF)], sems[b])

        def drain(b):
            pltpu.make_async_copy(
                emb_hbm.at[pl.ds(0, L)], rows_v.at[b], sems[b]).wait()

        def reduce_store(s, b):
            def rbody(i, accs):
                r = i * 8
                out = []
                for k in range(4):
                    v = [rows_v[b, r + j, pl.ds(LANES * k, LANES)]
                         for j in range(8)]
                    t = ((v[0] + v[1]) + (v[2] + v[3])) \
                        + ((v[4] + v[5]) + (v[6] + v[7]))
                    out.append(accs[k] + t)
                return tuple(out)

            zero = jnp.zeros((LANES,), jnp.float32)
            accs = lax.fori_loop(0, L // 8, rbody, (zero,) * 4)
            for k in range(4):
                out_v[s, pl.ds(LANES * k, LANES)] = accs[k]

        issue(0, 0)

        def body2(i, carry):
            s0 = 2 * i
            issue(s0 + 1, 1)
            drain(0)
            reduce_store(s0, 0)

            @pl.when(i + 1 < SPW // 2)
            def _():
                issue(s0 + 2, 0)

            drain(1)
            reduce_store(s0 + 1, 1)
            return carry

        lax.fori_loop(0, SPW // 2, body2, 0)
        pltpu.sync_copy(out_v, out_hbm.at[pl.ds(base, SPW)])

    return bag(x2, emb)


def _mlp_tc(sums, W1, b1, W2, b2):
    blk = 512

    def mlp_body(s_ref, w1_ref, b1_ref, w2_ref, b2_ref, o_ref):
        xa = s_ref[...] * (1.0 / L)
        h = jnp.dot(xa, w1_ref[...], preferred_element_type=jnp.float32)
        h = jnp.maximum(h + b1_ref[...], 0.0)
        o_ref[...] = (
            jnp.dot(h, w2_ref[...], preferred_element_type=jnp.float32)
            + b2_ref[...])

    return pl.pallas_call(
        mlp_body,
        grid=(B // blk,),
        in_specs=[
            pl.BlockSpec((blk, EMB), lambda i: (i, 0)),
            pl.BlockSpec((EMB, HID), lambda i: (0, 0)),
            pl.BlockSpec((1, HID), lambda i: (0, 0)),
            pl.BlockSpec((HID, TAGS), lambda i: (0, 0)),
            pl.BlockSpec((1, TAGS), lambda i: (0, 0)),
        ],
        out_specs=pl.BlockSpec((blk, TAGS), lambda i: (i, 0)),
        out_shape=jax.ShapeDtypeStruct((B, TAGS), jnp.float32),
    )(sums, W1, b1, W2, b2)


def kernel(x, emb, W1, b1, W2, b2):
    x2 = x.reshape(2 * B, HALF)
    sums = _embedding_bag_sc(x2, emb)
    return _mlp_tc(sums, W1, b1.reshape(1, HID), W2, b2.reshape(1, TAGS))


# speedup vs baseline: 1.1748x; 1.0037x over previous
import functools

import jax
import jax.numpy as jnp
from jax import lax
from jax.experimental import pallas as pl
from jax.experimental.pallas import tpu as pltpu
from jax.experimental.pallas import tpu_sc as plsc

EMB = 64
HID = 256
TAGS = 20
B = 4096
L = 200
HA = 104
HB = 96
LANES = 16
NC = 2
NS = 16
NW = NC * NS
SPW = B // NW


def _embedding_bag_sc(x, emb):
    mesh = plsc.VectorSubcoreMesh(core_axis_name="c", subcore_axis_name="s")

    @functools.partial(
        pl.kernel,
        out_type=jax.ShapeDtypeStruct((B, EMB), jnp.float32),
        mesh=mesh,
        compiler_params=pltpu.CompilerParams(use_tc_tiling_on_sc=False),
        scratch_types=[
            pltpu.VMEM((SPW, HA), jnp.int32),
            pltpu.VMEM((SPW, HB), jnp.int32),
            pltpu.VMEM((2, L, EMB), jnp.float32),
            pltpu.VMEM((SPW, EMB), jnp.float32),
            pltpu.SemaphoreType.DMA,
            pltpu.SemaphoreType.DMA,
        ],
    )
    def bag(x_hbm, emb_hbm, out_hbm, idxa_v, idxb_v, rows_v, out_v, sem0, sem1):
        wid = lax.axis_index("s") * NC + lax.axis_index("c")
        base = wid * SPW
        pltpu.sync_copy(x_hbm.at[pl.ds(base, SPW), pl.ds(0, HA)], idxa_v)
        pltpu.sync_copy(x_hbm.at[pl.ds(base, SPW), pl.ds(HA, HB)], idxb_v)
        sems = (sem0, sem1)

        def issue(s, b):
            pltpu.async_copy(
                emb_hbm.at[idxa_v.at[s]], rows_v.at[b, pl.ds(0, HA)], sems[b])
            pltpu.async_copy(
                emb_hbm.at[idxb_v.at[s]],
                rows_v.at[b, pl.ds(HA, HB)], sems[b])

        def drain(b):
            pltpu.make_async_copy(
                emb_hbm.at[pl.ds(0, L)], rows_v.at[b], sems[b]).wait()

        def reduce_store(s, b):
            def rbody(i, accs):
                r = i * 8
                out = []
                for k in range(4):
                    v = [rows_v[b, r + j, pl.ds(LANES * k, LANES)]
                         for j in range(8)]
                    t = ((v[0] + v[1]) + (v[2] + v[3])) \
                        + ((v[4] + v[5]) + (v[6] + v[7]))
                    out.append(accs[k] + t)
                return tuple(out)

            zero = jnp.zeros((LANES,), jnp.float32)
            accs = lax.fori_loop(0, L // 8, rbody, (zero,) * 4)
            for k in range(4):
                out_v[s, pl.ds(LANES * k, LANES)] = accs[k]

        issue(0, 0)

        def body2(i, carry):
            s0 = 2 * i
            issue(s0 + 1, 1)
            drain(0)
            reduce_store(s0, 0)

            @pl.when(i + 1 < SPW // 2)
            def _():
                issue(s0 + 2, 0)

            drain(1)
            reduce_store(s0 + 1, 1)
            return carry

        lax.fori_loop(0, SPW // 2, body2, 0)
        pltpu.sync_copy(out_v, out_hbm.at[pl.ds(base, SPW)])

    return bag(x, emb)


def _mlp_tc(sums, W1, b1, W2, b2):
    blk = 512

    def mlp_body(s_ref, w1_ref, b1_ref, w2_ref, b2_ref, o_ref):
        xa = s_ref[...] * (1.0 / L)
        h = jnp.dot(xa, w1_ref[...], preferred_element_type=jnp.float32)
        h = jnp.maximum(h + b1_ref[...], 0.0)
        o_ref[...] = (
            jnp.dot(h, w2_ref[...], preferred_element_type=jnp.float32)
            + b2_ref[...])

    return pl.pallas_call(
        mlp_body,
        grid=(B // blk,),
        in_specs=[
            pl.BlockSpec((blk, EMB), lambda i: (i, 0)),
            pl.BlockSpec((EMB, HID), lambda i: (0, 0)),
            pl.BlockSpec((1, HID), lambda i: (0, 0)),
            pl.BlockSpec((HID, TAGS), lambda i: (0, 0)),
            pl.BlockSpec((1, TAGS), lambda i: (0, 0)),
        ],
        out_specs=pl.BlockSpec((blk, TAGS), lambda i: (i, 0)),
        out_shape=jax.ShapeDtypeStruct((B, TAGS), jnp.float32),
    )(sums, W1, b1, W2, b2)


def kernel(x, emb, W1, b1, W2, b2):
    sums = _embedding_bag_sc(x, emb)
    return _mlp_tc(sums, W1, b1.reshape(1, HID), W2, b2.reshape(1, TAGS))
